# Initial kernel scaffold; baseline (speedup 1.0000x reference)
import jax
import jax.numpy as jnp
from jax.experimental import pallas as pl


def kernel(logits, labels, centers):
    raise NotImplementedError("write your pallas kernel here")


# speedup vs baseline: 6.3598x; 6.3598x over previous
import functools

import jax
import jax.numpy as jnp
from jax import lax
from jax.experimental import pallas as pl
from jax.experimental.pallas import tpu as pltpu
from jax.experimental.pallas import tpu_sc as plsc

NUM_CLASSES = 1000
K = NUM_CLASSES + 1
D = 128
B = 4096
KPAD = 1024

NC = 2
NS = 16
NW = NC * NS
BPW = B // NW

INTER_CLASS_WEIGHT = 0.001
EPS = 1e-05
DIAG_OFFSET = 100000.0


def _sc_intra_body(logits_hbm, labels_hbm, centers_hbm, out_hbm,
                   idx_v, rows_v, log_v, acc_v, sem):
    wid = lax.axis_index("s") * NC + lax.axis_index("c")
    base = wid * BPW

    pltpu.sync_copy(labels_hbm.at[pl.ds(base, BPW)], idx_v)
    pltpu.sync_copy(logits_hbm.at[pl.ds(base, BPW), :], log_v)
    pltpu.async_copy(centers_hbm.at[idx_v], rows_v, sem).wait()

    def row_body(i, acc):
        for c in range(D // 16):
            d = log_v[i, pl.ds(c * 16, 16)] - rows_v[i, pl.ds(c * 16, 16)]
            acc = acc + d * d
        return acc

    acc = lax.fori_loop(0, BPW, row_body, jnp.zeros((16,), jnp.float32))
    acc_v[...] = acc
    pltpu.sync_copy(acc_v, out_hbm.at[wid])


@jax.jit
def _sc_intra(logits, labels, centers):
    mesh = plsc.VectorSubcoreMesh(core_axis_name="c", subcore_axis_name="s")
    kern = functools.partial(
        pl.kernel,
        mesh=mesh,
        out_type=jax.ShapeDtypeStruct((NW, 16), jnp.float32),
        scratch_types=[
            pltpu.VMEM((BPW,), jnp.int32),
            pltpu.VMEM((BPW, D), jnp.float32),
            pltpu.VMEM((BPW, D), jnp.float32),
            pltpu.VMEM((16,), jnp.float32),
            pltpu.SemaphoreType.DMA,
        ],
    )(_sc_intra_body)
    return kern(logits, labels, centers)


def _tc_inter_body(c_ref, o_ref):
    c = c_ref[...]
    gram = lax.dot_general(c, c, (((1,), (1,)), ((), ())),
                           preferred_element_type=jnp.float32)
    n_row = jnp.sum(c * c, axis=1, keepdims=True)
    ones = jnp.ones((1, D), jnp.float32)
    n_col = lax.dot_general(ones, c * c, (((1,), (1,)), ((), ())),
                            preferred_element_type=jnp.float32)
    d2 = jnp.maximum(n_row + n_col - 2.0 * gram, 0.0)
    dist = jnp.sqrt(d2)

    row = lax.broadcasted_iota(jnp.int32, (KPAD, KPAD), 0)
    col = lax.broadcasted_iota(jnp.int32, (KPAD, KPAD), 1)
    valid = (row < K) & (col < K) & (row != col)
    terms = jnp.where(valid, 1.0 / (dist + EPS), 0.0)
    diag = K / (DIAG_OFFSET + EPS)
    o_ref[...] = jnp.sum(terms, keepdims=True) + diag


@jax.jit
def _tc_inter(centers_padded):
    return pl.pallas_call(
        _tc_inter_body,
        out_shape=jax.ShapeDtypeStruct((1, 1), jnp.float32),
    )(centers_padded)


def kernel(logits, labels, centers):
    labels = labels.astype(jnp.int32)
    partials = _sc_intra(logits, labels, centers)
    centers_padded = jnp.pad(centers, ((0, KPAD - K), (0, 0)))
    inter = _tc_inter(centers_padded)
    return partials.sum() / logits.shape[0] + INTER_CLASS_WEIGHT * inter[0, 0]


# speedup vs baseline: 6.6365x; 1.0435x over previous
import functools

import jax
import jax.numpy as jnp
from jax import lax
from jax.experimental import pallas as pl
from jax.experimental.pallas import tpu as pltpu
from jax.experimental.pallas import tpu_sc as plsc

NUM_CLASSES = 1000
K = NUM_CLASSES + 1
D = 128
B = 4096

NC = 2
NS = 16
NW = NC * NS
BPW = B // NW

INTER_CLASS_WEIGHT = 0.001
EPS = 1e-05
DIAG_OFFSET = 100000.0


def _sc_intra_body(logits_hbm, labels_hbm, centers_hbm, out_hbm,
                   idx_v, rows_v, log_v, acc_v, sem_log, sem_rows):
    wid = lax.axis_index("s") * NC + lax.axis_index("c")
    base = wid * BPW

    cp_log = pltpu.async_copy(logits_hbm.at[pl.ds(base, BPW), :], log_v,
                              sem_log)
    pltpu.sync_copy(labels_hbm.at[pl.ds(base, BPW)], idx_v)
    cp_rows = pltpu.async_copy(centers_hbm.at[idx_v], rows_v, sem_rows)
    cp_log.wait()
    cp_rows.wait()

    def row_body(i, carry):
        a0, a1 = carry
        r = i * 2
        for rr in (r, r + 1):
            for c in range(D // 16):
                d = log_v[rr, pl.ds(c * 16, 16)] - rows_v[rr, pl.ds(c * 16, 16)]
                if c % 2 == 0:
                    a0 = a0 + d * d
                else:
                    a1 = a1 + d * d
        return a0, a1

    zero = jnp.zeros((16,), jnp.float32)
    a0, a1 = lax.fori_loop(0, BPW // 2, row_body, (zero, zero))
    acc_v[...] = a0 + a1
    pltpu.sync_copy(acc_v, out_hbm.at[wid])


@jax.jit
def _sc_intra(logits, labels, centers):
    mesh = plsc.VectorSubcoreMesh(core_axis_name="c", subcore_axis_name="s")
    kern = functools.partial(
        pl.kernel,
        mesh=mesh,
        out_type=jax.ShapeDtypeStruct((NW, 16), jnp.float32),
        scratch_types=[
            pltpu.VMEM((BPW,), jnp.int32),
            pltpu.VMEM((BPW, D), jnp.float32),
            pltpu.VMEM((BPW, D), jnp.float32),
            pltpu.VMEM((16,), jnp.float32),
            pltpu.SemaphoreType.DMA,
            pltpu.SemaphoreType.DMA,
        ],
    )(_sc_intra_body)
    return kern(logits, labels, centers)


def _tc_inter_body(c_ref, o_ref):
    c = c_ref[...]
    gram = lax.dot_general(c, c, (((1,), (1,)), ((), ())),
                           preferred_element_type=jnp.float32)
    n_row = jnp.sum(c * c, axis=1, keepdims=True)
    ones = jnp.ones((1, D), jnp.float32)
    n_col = lax.dot_general(ones, c * c, (((1,), (1,)), ((), ())),
                            preferred_element_type=jnp.float32)
    d2 = jnp.maximum(n_row + n_col - 2.0 * gram, 0.0)
    dist = jnp.sqrt(d2)

    row = lax.broadcasted_iota(jnp.int32, (K, K), 0)
    col = lax.broadcasted_iota(jnp.int32, (K, K), 1)
    terms = jnp.where(row != col, 1.0 / (dist + EPS), 0.0)
    diag = K / (DIAG_OFFSET + EPS)
    o_ref[...] = jnp.sum(terms, keepdims=True) + diag


@jax.jit
def _tc_inter(centers):
    return pl.pallas_call(
        _tc_inter_body,
        out_shape=jax.ShapeDtypeStruct((1, 1), jnp.float32),
    )(centers)


def kernel(logits, labels, centers):
    labels = labels.astype(jnp.int32)
    partials = _sc_intra(logits, labels, centers)
    inter = _tc_inter(centers)
    return partials.sum() / logits.shape[0] + INTER_CLASS_WEIGHT * inter[0, 0]
